# baseline (speedup 1.0000x reference)
import jax
import jax.numpy as jnp
from jax import lax
from jax.experimental import pallas as pl
from jax.experimental.pallas import tpu as pltpu
from jax.experimental.pallas import tpu_sc as plsc

ROWS = 64
N = 32768
L = 16
NV = N // L
RADIX = 256
T = 4
NG = NV // T
TWORDS = RADIX * L
CWORDS = T * TWORDS
MININT = -2147483648
IDXMASK = N - 1

_info = plsc.get_sparse_core_info()
NC = _info.num_cores
NS = _info.num_subcores
NW = NC * NS
ROWS_PER_W = ROWS // NW


def _to_sortable(bits):
    s = lax.shift_right_arithmetic(bits, 31)
    return bits ^ (s | MININT)


def _from_sortable(u):
    s = lax.shift_right_arithmetic(u, 31)
    return u ^ (jnp.invert(s) | MININT)


def _sort_kernel(v_hbm, vals_hbm, idx_hbm, keys, idxf, idxi,
                 in_sem, vals_sem, idx_sem, *counters):
    wid = lax.axis_index("s") * NC + lax.axis_index("c")
    lane = lax.iota(jnp.int32, L)
    ones = jnp.ones((L,), jnp.int32)

    def load_payload(ref, is_f32, v):
        x = ref[pl.ds(v * L, L)]
        return plsc.bitcast(x, jnp.int32) if is_f32 else x

    def gather_key(idx):
        return plsc.bitcast(plsc.load_gather(keys, [idx]), jnp.int32)

    def cidx_of(payload):
        d = lax.shift_right_logical(payload, 15) & (RADIX - 1)
        return lax.shift_left(d, 4) | lane

    def pos_addr(q):
        return lax.shift_left(q & (NG - 1), 6) | \
            lax.shift_left(lax.shift_right_logical(q, 9) & (T - 1), 4) | \
            lax.shift_right_logical(q, 11)

    def zero_tables():
        def zero(i, c):
            for t in range(T):
                counters[t][pl.ds(i * L, L)] = jnp.zeros((L,), jnp.int32)
            return c
        lax.fori_loop(0, TWORDS // L, zero, 0, unroll=4)

    def hist_phase(src, src_f32):
        def hist(g, c):
            ps = [load_payload(src, src_f32, g * T + t) for t in range(T)]
            cidxs = [cidx_of(p) for p in ps]
            for t in range(T):
                plsc.addupdate_scatter(counters[t], [cidxs[t]], ones)
            return c
        lax.fori_loop(0, NG, hist, 0, unroll=4)

    def scan_phase():
        G = 4

        def scan(g, carry):
            cs = [[counters[t][pl.ds((g * G + u) * L, L)] for t in range(T)]
                  for u in range(G)]
            tots = [c[0] + c[1] + c[2] + c[3] for c in cs]
            incs = [plsc.cumsum(tot) for tot in tots]
            sums = [jnp.sum(tot) for tot in tots]
            for u in range(G):
                ex = incs[u] - tots[u] + carry
                for t in range(T):
                    counters[t][pl.ds((g * G + u) * L, L)] = ex
                    if t < T - 1:
                        ex = ex + cs[u][t]
                carry = carry + sums[u]
            return carry
        lax.fori_loop(0, RADIX // G, scan, jnp.int32(0))

    def run_pass(src, src_f32, dst, dst_f32, refill, last):
        zero_tables()
        hist_phase(src, src_f32)
        scan_phase()

        def perm(g, c):
            ps = [load_payload(src, src_f32, g * T + t) for t in range(T)]
            cidxs = [cidx_of(p) for p in ps]
            poss = [plsc.load_gather(counters[t], [cidxs[t]])
                    for t in range(T)]
            idxs = [p & IDXMASK for p in ps]
            if not last and refill:
                ks = [gather_key(i) for i in idxs]
            for t in range(T):
                plsc.store_scatter(counters[t], [cidxs[t]], poss[t] + ones)
            if last:
                for t in range(T):
                    plsc.store_scatter(dst, [poss[t]], idxs[t])
            else:
                if refill:
                    his = [lax.shift_left(
                        lax.shift_right_logical(k, 16), 15) for k in ks]
                else:
                    his = [lax.shift_left(
                        lax.shift_right_logical(p, 23), 15) for p in ps]
                addrs = [pos_addr(pos) for pos in poss]
                for t in range(T):
                    out = his[t] | idxs[t]
                    if dst_f32:
                        out = plsc.bitcast(out, jnp.float32)
                    plsc.store_scatter(dst, [addrs[t]], out)
            return c
        lax.fori_loop(0, NG, perm, 0, unroll=4)

    def do_row(r, c):
        row = wid * ROWS_PER_W + r
        pltpu.make_async_copy(v_hbm.at[row], keys, in_sem).wait()

        zero_tables()

        def fill(g, c2):
            jidxs = [lax.shift_left(lane, 11) | (t * NG + g)
                     for t in range(T)]
            raw = [plsc.bitcast(plsc.load_gather(keys, [ji]), jnp.int32)
                   for ji in jidxs]
            ks = [_to_sortable(b) for b in raw]
            cidxs = [lax.shift_left(k & (RADIX - 1), 4) | lane for k in ks]
            for t in range(T):
                plsc.store_scatter(keys, [jidxs[t]],
                                   plsc.bitcast(ks[t], jnp.float32))
            for t in range(T):
                plsc.addupdate_scatter(counters[t], [cidxs[t]], ones)
            return c2
        lax.fori_loop(0, NG, fill, 0, unroll=2)

        scan_phase()

        def perm0(g, c2):
            jidxs = [lax.shift_left(lane, 11) | (t * NG + g)
                     for t in range(T)]
            ks = [gather_key(ji) for ji in jidxs]
            cidxs = [lax.shift_left(k & (RADIX - 1), 4) | lane for k in ks]
            poss = [plsc.load_gather(counters[t], [cidxs[t]])
                    for t in range(T)]
            for t in range(T):
                plsc.store_scatter(counters[t], [cidxs[t]], poss[t] + ones)
            addrs = [pos_addr(p) for p in poss]
            outs = [lax.shift_left(lax.shift_right_logical(k, 8) & 0xFFFF, 15)
                    | ji for k, ji in zip(ks, jidxs)]
            for t in range(T):
                plsc.store_scatter(idxf, [addrs[t]],
                                   plsc.bitcast(outs[t], jnp.float32))
            return c2
        lax.fori_loop(0, NG, perm0, 0, unroll=4)

        run_pass(idxf, True, idxi, False, refill=True, last=False)
        run_pass(idxi, False, idxf, True, refill=False, last=False)
        run_pass(idxf, True, idxi, False, refill=False, last=True)


        pltpu.async_copy(idxi, idx_hbm.at[row], idx_sem)

        def vals(g, c2):
            vs = [g * T + t for t in range(T)]
            sidx = [idxi[pl.ds(v * L, L)] for v in vs]
            us = [gather_key(i) for i in sidx]
            for t in range(T):
                idxf[pl.ds(vs[t] * L, L)] = plsc.bitcast(
                    _from_sortable(us[t]), jnp.float32)
            return c2
        lax.fori_loop(0, NG, vals, 0, unroll=4)

        pltpu.async_copy(idxf, vals_hbm.at[row], vals_sem)

        @pl.when(r + 1 < ROWS_PER_W)
        def _():
            pltpu.async_copy(v_hbm.at[row + 1], keys, in_sem)

        pltpu.make_async_copy(idxi, idx_hbm.at[row], idx_sem).wait()
        pltpu.make_async_copy(idxf, vals_hbm.at[row], vals_sem).wait()
        return c

    pltpu.async_copy(v_hbm.at[wid * ROWS_PER_W], keys, in_sem)
    lax.fori_loop(0, ROWS_PER_W, do_row, 0)


@jax.jit
def kernel(v):
    mesh = plsc.VectorSubcoreMesh(core_axis_name="c", subcore_axis_name="s")
    f = pl.kernel(
        _sort_kernel,
        out_type=(
            jax.ShapeDtypeStruct((ROWS, N), jnp.float32),
            jax.ShapeDtypeStruct((ROWS, N), jnp.int32),
        ),
        mesh=mesh,
        scratch_types=[
            pltpu.VMEM((N,), jnp.float32),
            pltpu.VMEM((N,), jnp.float32),
            pltpu.VMEM((N,), jnp.int32),
            pltpu.SemaphoreType.DMA,
            pltpu.SemaphoreType.DMA,
            pltpu.SemaphoreType.DMA,
        ] + [pltpu.VMEM((TWORDS,), jnp.int32) for _ in range(T)],
        compiler_params=pltpu.CompilerParams(needs_layout_passes=False),
    )
    return f(v)


# speedup vs baseline: 1.2704x; 1.2704x over previous
import jax
import jax.numpy as jnp
from jax import lax
from jax.experimental import pallas as pl
from jax.experimental.pallas import tpu as pltpu
from jax.experimental.pallas import tpu_sc as plsc

ROWS = 64
N = 32768
L = 16
NV = N // L
RADIX = 256
T = 4
NG = NV // T
TWORDS = RADIX * L
CWORDS = T * TWORDS
MININT = -2147483648
IDXMASK = N - 1

_info = plsc.get_sparse_core_info()
NC = _info.num_cores
NS = _info.num_subcores
NW = NC * NS
ROWS_PER_W = ROWS // NW


def _to_sortable(bits):
    s = lax.shift_right_arithmetic(bits, 31)
    return bits ^ (s | MININT)


def _from_sortable(u):
    s = lax.shift_right_arithmetic(u, 31)
    return u ^ (jnp.invert(s) | MININT)


def _sort_kernel(v_hbm, vals_hbm, idx_hbm, keys, idxf, idxi,
                 in_sem, vals_sem, idx_sem, *counters):
    wid = lax.axis_index("s") * NC + lax.axis_index("c")
    lane = lax.iota(jnp.int32, L)
    ones = jnp.ones((L,), jnp.int32)

    def load_payload(ref, is_f32, v):
        x = ref[pl.ds(v * L, L)]
        return plsc.bitcast(x, jnp.int32) if is_f32 else x

    def gather_key(idx):
        return plsc.bitcast(plsc.load_gather(keys, [idx]), jnp.int32)

    def cidx_of(payload):
        d = lax.shift_right_logical(payload, 15) & (RADIX - 1)
        return lax.shift_left(d, 4) | lane

    def pos_addr(q):
        return lax.shift_left(q & (NG - 1), 6) | \
            lax.shift_left(lax.shift_right_logical(q, 9) & (T - 1), 4) | \
            lax.shift_right_logical(q, 11)

    def zero_tables():
        def zero(i, c):
            for t in range(T):
                counters[t][pl.ds(i * L, L)] = jnp.zeros((L,), jnp.int32)
            return c
        lax.fori_loop(0, TWORDS // L, zero, 0, unroll=4)

    def hist_phase(src, src_f32):
        def hist(g, c):
            ps = [load_payload(src, src_f32, g * T + t) for t in range(T)]
            cidxs = [cidx_of(p) for p in ps]
            for t in range(T):
                plsc.addupdate_scatter(counters[t], [cidxs[t]], ones)
            return c
        lax.fori_loop(0, NG, hist, 0, unroll=8)

    def scan_phase():
        G = 4

        def scan(g, carry):
            cs = [[counters[t][pl.ds((g * G + u) * L, L)] for t in range(T)]
                  for u in range(G)]
            tots = [c[0] + c[1] + c[2] + c[3] for c in cs]
            incs = [plsc.cumsum(tot) for tot in tots]
            sums = [jnp.sum(tot) for tot in tots]
            for u in range(G):
                ex = incs[u] - tots[u] + carry
                for t in range(T):
                    counters[t][pl.ds((g * G + u) * L, L)] = ex
                    if t < T - 1:
                        ex = ex + cs[u][t]
                carry = carry + sums[u]
            return carry
        lax.fori_loop(0, RADIX // G, scan, jnp.int32(0))

    def run_pass(src, src_f32, dst, dst_f32, refill, last):
        zero_tables()
        hist_phase(src, src_f32)
        scan_phase()

        def perm(g, c):
            ps = [load_payload(src, src_f32, g * T + t) for t in range(T)]
            cidxs = [cidx_of(p) for p in ps]
            poss = [plsc.load_gather(counters[t], [cidxs[t]])
                    for t in range(T)]
            idxs = [p & IDXMASK for p in ps]
            if not last and refill:
                ks = [gather_key(i) for i in idxs]
            for t in range(T):
                plsc.store_scatter(counters[t], [cidxs[t]], poss[t] + ones)
            if last:
                for t in range(T):
                    plsc.store_scatter(dst, [poss[t]], idxs[t])
            else:
                if refill:
                    his = [lax.shift_left(
                        lax.shift_right_logical(k, 16), 15) for k in ks]
                else:
                    his = [lax.shift_left(
                        lax.shift_right_logical(p, 23), 15) for p in ps]
                addrs = [pos_addr(pos) for pos in poss]
                for t in range(T):
                    out = his[t] | idxs[t]
                    if dst_f32:
                        out = plsc.bitcast(out, jnp.float32)
                    plsc.store_scatter(dst, [addrs[t]], out)
            return c
        lax.fori_loop(0, NG, perm, 0, unroll=8)

    def do_row(r, c):
        row = wid * ROWS_PER_W + r
        pltpu.make_async_copy(v_hbm.at[row], keys, in_sem).wait()

        def fill(g, c2):
            vs = [g * T + t for t in range(T)]
            bs = [plsc.bitcast(keys[pl.ds(v * L, L)], jnp.int32)
                  for v in vs]
            ks = [_to_sortable(b) for b in bs]
            for t in range(T):
                keys[pl.ds(vs[t] * L, L)] = plsc.bitcast(ks[t], jnp.float32)
            js = [v * L + lane for v in vs]
            pls = [lax.shift_left(k & 0xFFFF, 15) | j
                   for k, j in zip(ks, js)]
            addrs = [pos_addr(j) for j in js]
            for t in range(T):
                plsc.store_scatter(idxi, [addrs[t]], pls[t])
            return c2
        lax.fori_loop(0, NG, fill, 0, unroll=4)

        run_pass(idxi, False, idxf, True, refill=False, last=False)
        run_pass(idxf, True, idxi, False, refill=True, last=False)
        run_pass(idxi, False, idxf, True, refill=False, last=False)
        run_pass(idxf, True, idxi, False, refill=False, last=True)

        pltpu.async_copy(idxi, idx_hbm.at[row], idx_sem)

        def vals(g, c2):
            vs = [g * T + t for t in range(T)]
            sidx = [idxi[pl.ds(v * L, L)] for v in vs]
            us = [gather_key(i) for i in sidx]
            for t in range(T):
                idxf[pl.ds(vs[t] * L, L)] = plsc.bitcast(
                    _from_sortable(us[t]), jnp.float32)
            return c2
        lax.fori_loop(0, NG, vals, 0, unroll=8)

        pltpu.async_copy(idxf, vals_hbm.at[row], vals_sem)

        @pl.when(r + 1 < ROWS_PER_W)
        def _():
            pltpu.async_copy(v_hbm.at[row + 1], keys, in_sem)

        pltpu.make_async_copy(idxi, idx_hbm.at[row], idx_sem).wait()
        pltpu.make_async_copy(idxf, vals_hbm.at[row], vals_sem).wait()
        return c

    pltpu.async_copy(v_hbm.at[wid * ROWS_PER_W], keys, in_sem)
    lax.fori_loop(0, ROWS_PER_W, do_row, 0)


@jax.jit
def kernel(v):
    mesh = plsc.VectorSubcoreMesh(core_axis_name="c", subcore_axis_name="s")
    f = pl.kernel(
        _sort_kernel,
        out_type=(
            jax.ShapeDtypeStruct((ROWS, N), jnp.float32),
            jax.ShapeDtypeStruct((ROWS, N), jnp.int32),
        ),
        mesh=mesh,
        scratch_types=[
            pltpu.VMEM((N,), jnp.float32),
            pltpu.VMEM((N,), jnp.float32),
            pltpu.VMEM((N,), jnp.int32),
            pltpu.SemaphoreType.DMA,
            pltpu.SemaphoreType.DMA,
            pltpu.SemaphoreType.DMA,
        ] + [pltpu.VMEM((TWORDS,), jnp.int32) for _ in range(T)],
        compiler_params=pltpu.CompilerParams(needs_layout_passes=False),
    )
    return f(v)


# speedup vs baseline: 1.4692x; 1.1565x over previous
import jax
import jax.numpy as jnp
from jax import lax
from jax.experimental import pallas as pl
from jax.experimental.pallas import tpu as pltpu
from jax.experimental.pallas import tpu_sc as plsc

ROWS = 64
N = 32768
L = 16
NV = N // L
RADIX = 256
T = 4
NG = NV // T
TWORDS = RADIX * L
CWORDS = T * TWORDS
MININT = -2147483648
IDXMASK = N - 1

_info = plsc.get_sparse_core_info()
NC = _info.num_cores
NS = _info.num_subcores
NW = NC * NS
ROWS_PER_W = ROWS // NW


def _to_sortable(bits):
    s = lax.shift_right_arithmetic(bits, 31)
    return bits ^ (s | MININT)


def _from_sortable(u):
    s = lax.shift_right_arithmetic(u, 31)
    return u ^ (jnp.invert(s) | MININT)


def _sort_kernel(v_hbm, vals_hbm, idx_hbm, keys, idxf, idxi,
                 in_sem, vals_sem, idx_sem, *counters):
    wid = lax.axis_index("s") * NC + lax.axis_index("c")
    lane = lax.iota(jnp.int32, L)
    ones = jnp.ones((L,), jnp.int32)

    def load_payload(ref, is_f32, v):
        x = ref[pl.ds(v * L, L)]
        return plsc.bitcast(x, jnp.int32) if is_f32 else x

    def gather_key(idx):
        return plsc.bitcast(plsc.load_gather(keys, [idx]), jnp.int32)

    def cidx_of(payload):
        d = lax.shift_right_logical(payload, 15) & (RADIX - 1)
        return lax.shift_left(d, 4) | lane

    def pos_addr(q):
        return lax.shift_left(q & (NV - 1), 4) | \
            lax.shift_right_logical(q, 11)

    def zero_tables():
        def zero(i, c):
            for t in range(T):
                counters[t][pl.ds(i * L, L)] = jnp.zeros((L,), jnp.int32)
            return c
        lax.fori_loop(0, TWORDS // L, zero, 0, unroll=4)

    def hist_phase(src, src_f32):
        def hist(g, c):
            ps = [load_payload(src, src_f32, t * NG + g) for t in range(T)]
            cidxs = [cidx_of(p) for p in ps]
            for t in range(T):
                plsc.addupdate_scatter(counters[t], [cidxs[t]], ones)
            return c
        lax.fori_loop(0, NG, hist, 0, unroll=4)

    def scan_phase():
        G = 4

        def scan(g, carry):
            cs = [[counters[t][pl.ds((g * G + u) * L, L)] for t in range(T)]
                  for u in range(G)]
            tots = [c[0] + c[1] + c[2] + c[3] for c in cs]
            incs = [plsc.cumsum(tot) for tot in tots]
            sums = [jnp.sum(tot) for tot in tots]
            for u in range(G):
                ex = incs[u] - tots[u] + carry
                for t in range(T):
                    counters[t][pl.ds((g * G + u) * L, L)] = ex
                    if t < T - 1:
                        ex = ex + cs[u][t]
                carry = carry + sums[u]
            return carry
        lax.fori_loop(0, RADIX // G, scan, jnp.int32(0))

    def run_pass(src, src_f32, dst, dst_f32, refill, last):
        zero_tables()
        hist_phase(src, src_f32)
        scan_phase()

        def perm(g, c):
            ps = [load_payload(src, src_f32, t * NG + g) for t in range(T)]
            cidxs = [cidx_of(p) for p in ps]
            poss = [plsc.load_gather(counters[t], [cidxs[t]])
                    for t in range(T)]
            idxs = [p & IDXMASK for p in ps]
            if not last and refill:
                ks = [gather_key(i) for i in idxs]
            for t in range(T):
                plsc.store_scatter(counters[t], [cidxs[t]], poss[t] + ones)
            if last:
                for t in range(T):
                    plsc.store_scatter(dst, [poss[t]], idxs[t])
            else:
                if refill:
                    his = [lax.shift_left(
                        lax.shift_right_logical(k, 16), 15) for k in ks]
                else:
                    his = [lax.shift_left(
                        lax.shift_right_logical(p, 23), 15) for p in ps]
                addrs = [pos_addr(pos) for pos in poss]
                for t in range(T):
                    out = his[t] | idxs[t]
                    if dst_f32:
                        out = plsc.bitcast(out, jnp.float32)
                    plsc.store_scatter(dst, [addrs[t]], out)
            return c
        lax.fori_loop(0, NG, perm, 0, unroll=4)

    def do_row(r, c):
        row = wid * ROWS_PER_W + r
        pltpu.make_async_copy(v_hbm.at[row], keys, in_sem).wait()

        def fill(g, c2):
            vs = [g * T + t for t in range(T)]
            bs = [plsc.bitcast(keys[pl.ds(v * L, L)], jnp.int32)
                  for v in vs]
            ks = [_to_sortable(b) for b in bs]
            for t in range(T):
                keys[pl.ds(vs[t] * L, L)] = plsc.bitcast(ks[t], jnp.float32)
            js = [v * L + lane for v in vs]
            pls = [lax.shift_left(k & 0xFFFF, 15) | j
                   for k, j in zip(ks, js)]
            addrs = [pos_addr(j) for j in js]
            for t in range(T):
                plsc.store_scatter(idxi, [addrs[t]], pls[t])
            return c2
        lax.fori_loop(0, NG, fill, 0, unroll=2)

        run_pass(idxi, False, idxf, True, refill=False, last=False)
        run_pass(idxf, True, idxi, False, refill=True, last=False)
        run_pass(idxi, False, idxf, True, refill=False, last=False)
        run_pass(idxf, True, idxi, False, refill=False, last=True)

        pltpu.async_copy(idxi, idx_hbm.at[row], idx_sem)

        def vals(g, c2):
            vs = [g * T + t for t in range(T)]
            sidx = [idxi[pl.ds(v * L, L)] for v in vs]
            us = [gather_key(i) for i in sidx]
            for t in range(T):
                idxf[pl.ds(vs[t] * L, L)] = plsc.bitcast(
                    _from_sortable(us[t]), jnp.float32)
            return c2
        lax.fori_loop(0, NG, vals, 0, unroll=4)

        pltpu.async_copy(idxf, vals_hbm.at[row], vals_sem)

        @pl.when(r + 1 < ROWS_PER_W)
        def _():
            pltpu.async_copy(v_hbm.at[row + 1], keys, in_sem)

        pltpu.make_async_copy(idxi, idx_hbm.at[row], idx_sem).wait()
        pltpu.make_async_copy(idxf, vals_hbm.at[row], vals_sem).wait()
        return c

    pltpu.async_copy(v_hbm.at[wid * ROWS_PER_W], keys, in_sem)
    lax.fori_loop(0, ROWS_PER_W, do_row, 0)


@jax.jit
def kernel(v):
    mesh = plsc.VectorSubcoreMesh(core_axis_name="c", subcore_axis_name="s")
    f = pl.kernel(
        _sort_kernel,
        out_type=(
            jax.ShapeDtypeStruct((ROWS, N), jnp.float32),
            jax.ShapeDtypeStruct((ROWS, N), jnp.int32),
        ),
        mesh=mesh,
        scratch_types=[
            pltpu.VMEM((N,), jnp.float32),
            pltpu.VMEM((N,), jnp.float32),
            pltpu.VMEM((N,), jnp.int32),
            pltpu.SemaphoreType.DMA,
            pltpu.SemaphoreType.DMA,
            pltpu.SemaphoreType.DMA,
        ] + [pltpu.VMEM((TWORDS,), jnp.int32) for _ in range(T)],
        compiler_params=pltpu.CompilerParams(needs_layout_passes=False),
    )
    return f(v)
